# baseline (speedup 1.0000x reference)
import jax
import jax.numpy as jnp
from jax import lax
from jax.experimental import pallas as pl
from jax.experimental.pallas import tpu as pltpu
from jax.experimental.pallas import tpu_sc as plsc

B = 4096
L = 200
VOCAB = 1000000
EMBED_DIM = 64
PAD_DIM = 128

_info = plsc.get_sparse_core_info()
_NC = _info.num_cores
_NS = _info.num_subcores
_NW = _NC * _NS

_TC = 256
_NFULL = VOCAB // _TC
_KPW = _NFULL // _NW
_NEXTRA = _NFULL - _KPW * _NW
_TAIL = VOCAB - _NFULL * _TC


def _transpose_kernel(wt_hbm, out_hbm, in_v0, in_v1, out_v0, out_v1, in_t, out_t,
                      sem_i0, sem_i1, sem_o0, sem_o1, sem_t):
    in_v = (in_v0, in_v1)
    out_v = (out_v0, out_v1)
    sem_i = (sem_i0, sem_i1)
    sem_o = (sem_o0, sem_o1)
    wid = lax.axis_index("s") * _NC + lax.axis_index("c")

    iota16 = lax.iota(jnp.int32, 16)
    iota64 = iota16 * EMBED_DIM

    def in_start(start, b):
        pltpu.async_copy(wt_hbm.at[:, pl.ds(start, _TC)], in_v[b], sem_i[b])

    def in_wait(start, b):
        pltpu.make_async_copy(
            wt_hbm.at[:, pl.ds(start, _TC)], in_v[b], sem_i[b]
        ).wait()

    def out_start(start, b):
        pltpu.async_copy(
            out_v[b], out_hbm.at[pl.ds(start * EMBED_DIM, _TC * EMBED_DIM)],
            sem_o[b],
        )

    def out_wait(start, b):
        pltpu.make_async_copy(
            out_v[b], out_hbm.at[pl.ds(start * EMBED_DIM, _TC * EMBED_DIM)],
            sem_o[b],
        ).wait()

    def transpose_body(b):
        @pl.loop(0, _TC // 16)
        def _(tb):
            t0 = tb * 16
            col_idx = iota16 + t0
            for j2 in range(0, 16, 2):
                batch = []
                for j in (j2, j2 + 1):
                    rot_j = (iota16 + j) & 15
                    out_j = iota64 + rot_j
                    for d0 in range(0, EMBED_DIM, 16):
                        v = plsc.load_gather(
                            in_v[b].at[pl.ds(d0, 16)], [rot_j, col_idx]
                        )
                        batch.append((out_j + (t0 * EMBED_DIM + d0), v))
                for oidx, v in batch:
                    plsc.store_scatter(out_v[b], [oidx], v)

    def chunk_start(c):
        return c * _TC

    in_start(chunk_start(wid), 0)
    in_start(chunk_start(_NW + wid), 1)

    @pl.loop(0, _KPW - 2, step=2)
    def _(k):
        for b in range(2):
            c = (k + b) * _NW + wid
            start = chunk_start(c)
            in_wait(start, b)

            @pl.when(k + b >= 2)
            def _():
                out_wait(chunk_start((k + b - 2) * _NW + wid), b)

            transpose_body(b)
            out_start(start, b)
            in_start(chunk_start((k + b + 2) * _NW + wid), b)

    for b in range(2):
        c = (_KPW - 2 + b) * _NW + wid
        start = chunk_start(c)
        in_wait(start, b)
        out_wait(chunk_start((_KPW - 4 + b) * _NW + wid), b)
        transpose_body(b)
        out_start(start, b)

    @pl.when(wid < _NEXTRA)
    def _():
        start = chunk_start(_KPW * _NW + wid)
        in_start(start, 0)
        in_wait(start, 0)
        out_wait(chunk_start((_KPW - 2) * _NW + wid), 0)
        transpose_body(0)
        out_start(start, 0)
        out_wait(start, 0)

    @pl.when(wid >= _NEXTRA)
    def _():
        out_wait(chunk_start((_KPW - 2) * _NW + wid), 0)

    out_wait(chunk_start((_KPW - 1) * _NW + wid), 1)

    @pl.when(wid == _NEXTRA)
    def _():
        start = _NFULL * _TC
        pltpu.async_copy(wt_hbm.at[:, pl.ds(start, _TAIL)], in_t, sem_t)
        pltpu.make_async_copy(
            wt_hbm.at[:, pl.ds(start, _TAIL)], in_t, sem_t
        ).wait()

        @pl.loop(0, _TAIL // 16)
        def _(tb):
            t0 = tb * 16
            col_idx = iota16 + t0
            for j2 in range(0, 16, 2):
                batch = []
                for j in (j2, j2 + 1):
                    rot_j = (iota16 + j) & 15
                    out_j = iota64 + rot_j
                    for d0 in range(0, EMBED_DIM, 16):
                        v = plsc.load_gather(
                            in_t.at[pl.ds(d0, 16)], [rot_j, col_idx]
                        )
                        batch.append((out_j + (t0 * EMBED_DIM + d0), v))
                for oidx, v in batch:
                    plsc.store_scatter(out_t, [oidx], v)

        pltpu.async_copy(
            out_t, out_hbm.at[pl.ds(start * EMBED_DIM, _TAIL * EMBED_DIM)], sem_t
        )
        pltpu.make_async_copy(
            out_t, out_hbm.at[pl.ds(start * EMBED_DIM, _TAIL * EMBED_DIM)], sem_t
        ).wait()


_N = B * L
_TB = 128
_NBB = B // _TB
_NTASK = L * _NBB
_TPW = _NTASK // _NW


def _gather_t_kernel(idxt_hbm, table_hbm, out_hbm,
                     idx_v0, idx_v1, rows_v0, rows_v1, slab_v0, slab_v1,
                     sem_i0, sem_i1, sem_g0, sem_g1, sem_o0, sem_o1):
    idx_v = (idx_v0, idx_v1)
    rows_v = (rows_v0, rows_v1)
    slab_v = (slab_v0, slab_v1)
    sem_i = (sem_i0, sem_i1)
    sem_g = (sem_g0, sem_g1)
    sem_o = (sem_o0, sem_o1)
    wid = lax.axis_index("s") * _NC + lax.axis_index("c")
    iota16 = lax.iota(jnp.int32, 16)

    def idx_off(t):
        return (t // _NBB) * B + (t % _NBB) * _TB

    def idx_start(t, b):
        pltpu.async_copy(idxt_hbm.at[pl.ds(idx_off(t), _TB)], idx_v[b], sem_i[b])

    def idx_wait(t, b):
        pltpu.make_async_copy(
            idxt_hbm.at[pl.ds(idx_off(t), _TB)], idx_v[b], sem_i[b]
        ).wait()

    def gather_start(b):
        pltpu.async_copy(table_hbm.at[idx_v[b]], rows_v[b], sem_g[b])

    def gather_wait(b):
        pltpu.make_async_copy(table_hbm.at[idx_v[b]], rows_v[b], sem_g[b]).wait()

    def out_ref(t):
        return out_hbm.at[t // _NBB, :, t % _NBB]

    def out_start(t, b):
        pltpu.async_copy(slab_v[b], out_ref(t), sem_o[b])

    def out_wait(t, b):
        pltpu.make_async_copy(slab_v[b], out_ref(t), sem_o[b]).wait()

    def transpose_body(b):
        @pl.loop(0, _TB // 16)
        def _(tb):
            t_idx = iota16 + tb * 16
            for j2 in range(0, 16, 2):
                batch = []
                for j in (j2, j2 + 1):
                    rot_j = (iota16 + j) & 15
                    db_j = rot_j >> 3
                    d8_j = rot_j & 7
                    for d0 in range(0, EMBED_DIM, 16):
                        v = plsc.load_gather(rows_v[b], [t_idx, rot_j + d0])
                        batch.append((db_j + (d0 >> 3), d8_j, v))
                for idx_db, idx_d8, v in batch:
                    plsc.store_scatter(slab_v[b], [idx_db, idx_d8, t_idx], v)

    base = wid * _TPW
    for b in range(2):
        idx_start(base + b, b)
        idx_wait(base + b, b)
        gather_start(b)

    @pl.loop(0, _TPW - 2, step=2)
    def _(k):
        for b in range(2):
            t = base + k + b
            gather_wait(b)

            @pl.when(k + b >= 2)
            def _():
                out_wait(t - 2, b)

            transpose_body(b)
            out_start(t, b)
            idx_start(t + 2, b)
            idx_wait(t + 2, b)
            gather_start(b)

    for b in range(2):
        t = base + _TPW - 2 + b
        gather_wait(b)
        out_wait(t - 2, b)
        transpose_body(b)
        out_start(t, b)
    for b in range(2):
        out_wait(base + _TPW - 2 + b, b)


@jax.jit
def kernel(tokens, word_embed_weight):
    mesh = plsc.VectorSubcoreMesh(core_axis_name="c", subcore_axis_name="s")

    wt = word_embed_weight.T
    tlin = pl.kernel(
        _transpose_kernel,
        out_type=jax.ShapeDtypeStruct((VOCAB * EMBED_DIM,), jnp.float32),
        mesh=mesh,
        scratch_types=[
            pltpu.VMEM((EMBED_DIM, _TC), jnp.float32),
            pltpu.VMEM((EMBED_DIM, _TC), jnp.float32),
            pltpu.VMEM((_TC * EMBED_DIM,), jnp.float32),
            pltpu.VMEM((_TC * EMBED_DIM,), jnp.float32),
            pltpu.VMEM((EMBED_DIM, _TAIL), jnp.float32),
            pltpu.VMEM((_TAIL * EMBED_DIM,), jnp.float32),
            pltpu.SemaphoreType.DMA,
            pltpu.SemaphoreType.DMA,
            pltpu.SemaphoreType.DMA,
            pltpu.SemaphoreType.DMA,
            pltpu.SemaphoreType.DMA,
        ],
        compiler_params=pltpu.CompilerParams(
            use_tc_tiling_on_sc=True, needs_layout_passes=False
        ),
    )(wt)
    table = tlin.reshape(VOCAB, EMBED_DIM)

    idxt = tokens.T.reshape(_N).astype(jnp.int32)
    out = pl.kernel(
        _gather_t_kernel,
        out_type=jax.ShapeDtypeStruct(
            (L, EMBED_DIM // 8, B // _TB, 8, _TB), jnp.float32
        ),
        mesh=mesh,
        scratch_types=[
            pltpu.VMEM((_TB,), jnp.int32),
            pltpu.VMEM((_TB,), jnp.int32),
            pltpu.VMEM((_TB, EMBED_DIM), jnp.float32),
            pltpu.VMEM((_TB, EMBED_DIM), jnp.float32),
            pltpu.VMEM((EMBED_DIM // 8, 8, _TB), jnp.float32),
            pltpu.VMEM((EMBED_DIM // 8, 8, _TB), jnp.float32),
            pltpu.SemaphoreType.DMA,
            pltpu.SemaphoreType.DMA,
            pltpu.SemaphoreType.DMA,
            pltpu.SemaphoreType.DMA,
            pltpu.SemaphoreType.DMA,
            pltpu.SemaphoreType.DMA,
        ],
        compiler_params=pltpu.CompilerParams(
            use_tc_tiling_on_sc=False, needs_layout_passes=False
        ),
    )(idxt, table)
    return jnp.transpose(out, (2, 4, 0, 1, 3)).reshape(B, L, EMBED_DIM)


# speedup vs baseline: 1.1591x; 1.1591x over previous
import jax
import jax.numpy as jnp
from jax import lax
from jax.experimental import pallas as pl
from jax.experimental.pallas import tpu as pltpu
from jax.experimental.pallas import tpu_sc as plsc

B = 4096
L = 200
VOCAB = 1000000
EMBED_DIM = 64
PAD_DIM = 128

_info = plsc.get_sparse_core_info()
_NC = _info.num_cores
_NS = _info.num_subcores
_NW = _NC * _NS

_TC = 256
_NFULL = VOCAB // _TC
_KPW = _NFULL // _NW
_NEXTRA = _NFULL - _KPW * _NW
_TAIL = VOCAB - _NFULL * _TC


def _transpose_kernel(wt_hbm, out_hbm, in_v0, in_v1, out_v0, out_v1, in_t, out_t,
                      sem_i0, sem_i1, sem_o0, sem_o1, sem_t):
    in_v = (in_v0, in_v1)
    out_v = (out_v0, out_v1)
    sem_i = (sem_i0, sem_i1)
    sem_o = (sem_o0, sem_o1)
    wid = lax.axis_index("s") * _NC + lax.axis_index("c")

    iota16 = lax.iota(jnp.int32, 16)
    iota64 = iota16 * EMBED_DIM

    def in_start(start, b):
        pltpu.async_copy(wt_hbm.at[:, pl.ds(start, _TC)], in_v[b], sem_i[b])

    def in_wait(start, b):
        pltpu.make_async_copy(
            wt_hbm.at[:, pl.ds(start, _TC)], in_v[b], sem_i[b]
        ).wait()

    def out_start(start, b):
        pltpu.async_copy(
            out_v[b], out_hbm.at[pl.ds(start * EMBED_DIM, _TC * EMBED_DIM)],
            sem_o[b],
        )

    def out_wait(start, b):
        pltpu.make_async_copy(
            out_v[b], out_hbm.at[pl.ds(start * EMBED_DIM, _TC * EMBED_DIM)],
            sem_o[b],
        ).wait()

    def transpose_body(b):
        @pl.loop(0, _TC // 16)
        def _(tb):
            t0 = tb * 16
            col_idx = iota16 + t0
            for j2 in range(0, 16, 2):
                batch = []
                for j in (j2, j2 + 1):
                    rot_j = (iota16 + j) & 15
                    out_j = iota64 + rot_j
                    for d0 in range(0, EMBED_DIM, 16):
                        v = plsc.load_gather(
                            in_v[b].at[pl.ds(d0, 16)], [rot_j, col_idx]
                        )
                        batch.append((out_j + (t0 * EMBED_DIM + d0), v))
                for oidx, v in batch:
                    plsc.store_scatter(out_v[b], [oidx], v)

    def chunk_start(c):
        return c * _TC

    in_start(chunk_start(wid), 0)
    in_start(chunk_start(_NW + wid), 1)

    @pl.loop(0, _KPW - 2, step=2)
    def _(k):
        for b in range(2):
            c = (k + b) * _NW + wid
            start = chunk_start(c)
            in_wait(start, b)

            @pl.when(k + b >= 2)
            def _():
                out_wait(chunk_start((k + b - 2) * _NW + wid), b)

            transpose_body(b)
            out_start(start, b)
            in_start(chunk_start((k + b + 2) * _NW + wid), b)

    for b in range(2):
        c = (_KPW - 2 + b) * _NW + wid
        start = chunk_start(c)
        in_wait(start, b)
        out_wait(chunk_start((_KPW - 4 + b) * _NW + wid), b)
        transpose_body(b)
        out_start(start, b)

    @pl.when(wid < _NEXTRA)
    def _():
        start = chunk_start(_KPW * _NW + wid)
        in_start(start, 0)
        in_wait(start, 0)
        out_wait(chunk_start((_KPW - 2) * _NW + wid), 0)
        transpose_body(0)
        out_start(start, 0)
        out_wait(start, 0)

    @pl.when(wid >= _NEXTRA)
    def _():
        out_wait(chunk_start((_KPW - 2) * _NW + wid), 0)

    out_wait(chunk_start((_KPW - 1) * _NW + wid), 1)

    @pl.when(wid == _NEXTRA)
    def _():
        start = _NFULL * _TC
        pltpu.async_copy(wt_hbm.at[:, pl.ds(start, _TAIL)], in_t, sem_t)
        pltpu.make_async_copy(
            wt_hbm.at[:, pl.ds(start, _TAIL)], in_t, sem_t
        ).wait()

        @pl.loop(0, _TAIL // 16)
        def _(tb):
            t0 = tb * 16
            col_idx = iota16 + t0
            for j2 in range(0, 16, 2):
                batch = []
                for j in (j2, j2 + 1):
                    rot_j = (iota16 + j) & 15
                    out_j = iota64 + rot_j
                    for d0 in range(0, EMBED_DIM, 16):
                        v = plsc.load_gather(
                            in_t.at[pl.ds(d0, 16)], [rot_j, col_idx]
                        )
                        batch.append((out_j + (t0 * EMBED_DIM + d0), v))
                for oidx, v in batch:
                    plsc.store_scatter(out_t, [oidx], v)

        pltpu.async_copy(
            out_t, out_hbm.at[pl.ds(start * EMBED_DIM, _TAIL * EMBED_DIM)], sem_t
        )
        pltpu.make_async_copy(
            out_t, out_hbm.at[pl.ds(start * EMBED_DIM, _TAIL * EMBED_DIM)], sem_t
        ).wait()


_N = B * L
_TB = 128
_NBB = B // _TB
_NTASK = L * _NBB
_TPW = _NTASK // _NW


def _gather_t_kernel(idxt_hbm, table_hbm, out_hbm,
                     idx_all, rows_v0, rows_v1, slab_v0, slab_v1,
                     sem_g0, sem_g1, sem_o0, sem_o1):
    rows_v = (rows_v0, rows_v1)
    slab_v = (slab_v0, slab_v1)
    sem_g = (sem_g0, sem_g1)
    sem_o = (sem_o0, sem_o1)
    wid = lax.axis_index("s") * _NC + lax.axis_index("c")
    iota16 = lax.iota(jnp.int32, 16)

    pltpu.sync_copy(
        idxt_hbm.at[pl.ds(wid * _TPW * _TB, _TPW * _TB)], idx_all
    )

    def gather_start(k, b):
        pltpu.async_copy(
            table_hbm.at[idx_all.at[pl.ds(k * _TB, _TB)]], rows_v[b], sem_g[b]
        )

    def gather_wait(k, b):
        pltpu.make_async_copy(
            table_hbm.at[idx_all.at[pl.ds(k * _TB, _TB)]], rows_v[b], sem_g[b]
        ).wait()

    def out_ref(t):
        return out_hbm.at[t // _NBB, :, t % _NBB]

    def out_start(t, b):
        pltpu.async_copy(slab_v[b], out_ref(t), sem_o[b])

    def out_wait(t, b):
        pltpu.make_async_copy(slab_v[b], out_ref(t), sem_o[b]).wait()

    def transpose_body(b):
        @pl.loop(0, _TB // 16)
        def _(tb):
            t_idx = iota16 + tb * 16
            for j2 in range(0, 16, 2):
                batch = []
                for j in (j2, j2 + 1):
                    rot_j = (iota16 + j) & 15
                    db_j = rot_j >> 3
                    d8_j = rot_j & 7
                    for d0 in range(0, EMBED_DIM, 16):
                        v = plsc.load_gather(rows_v[b], [t_idx, rot_j + d0])
                        batch.append((db_j + (d0 >> 3), d8_j, v))
                for idx_db, idx_d8, v in batch:
                    plsc.store_scatter(slab_v[b], [idx_db, idx_d8, t_idx], v)

    base = wid * _TPW
    for b in range(2):
        gather_start(b, b)

    @pl.loop(0, _TPW - 2, step=2)
    def _(k):
        for b in range(2):
            t = base + k + b
            gather_wait(k + b, b)

            @pl.when(k + b >= 2)
            def _():
                out_wait(t - 2, b)

            transpose_body(b)
            out_start(t, b)
            gather_start(k + b + 2, b)

    for b in range(2):
        t = base + _TPW - 2 + b
        gather_wait(_TPW - 2 + b, b)
        out_wait(t - 2, b)
        transpose_body(b)
        out_start(t, b)
    for b in range(2):
        out_wait(base + _TPW - 2 + b, b)


@jax.jit
def kernel(tokens, word_embed_weight):
    mesh = plsc.VectorSubcoreMesh(core_axis_name="c", subcore_axis_name="s")

    wt = word_embed_weight.T
    tlin = pl.kernel(
        _transpose_kernel,
        out_type=jax.ShapeDtypeStruct((VOCAB * EMBED_DIM,), jnp.float32),
        mesh=mesh,
        scratch_types=[
            pltpu.VMEM((EMBED_DIM, _TC), jnp.float32),
            pltpu.VMEM((EMBED_DIM, _TC), jnp.float32),
            pltpu.VMEM((_TC * EMBED_DIM,), jnp.float32),
            pltpu.VMEM((_TC * EMBED_DIM,), jnp.float32),
            pltpu.VMEM((EMBED_DIM, _TAIL), jnp.float32),
            pltpu.VMEM((_TAIL * EMBED_DIM,), jnp.float32),
            pltpu.SemaphoreType.DMA,
            pltpu.SemaphoreType.DMA,
            pltpu.SemaphoreType.DMA,
            pltpu.SemaphoreType.DMA,
            pltpu.SemaphoreType.DMA,
        ],
        compiler_params=pltpu.CompilerParams(
            use_tc_tiling_on_sc=True, needs_layout_passes=False
        ),
    )(wt)
    table = tlin.reshape(VOCAB, EMBED_DIM)

    idxt = tokens.T.reshape(_N).astype(jnp.int32)
    out = pl.kernel(
        _gather_t_kernel,
        out_type=jax.ShapeDtypeStruct(
            (L, EMBED_DIM // 8, B // _TB, 8, _TB), jnp.float32
        ),
        mesh=mesh,
        scratch_types=[
            pltpu.VMEM((_TPW * _TB,), jnp.int32),
            pltpu.VMEM((_TB, EMBED_DIM), jnp.float32),
            pltpu.VMEM((_TB, EMBED_DIM), jnp.float32),
            pltpu.VMEM((EMBED_DIM // 8, 8, _TB), jnp.float32),
            pltpu.VMEM((EMBED_DIM // 8, 8, _TB), jnp.float32),
            pltpu.SemaphoreType.DMA,
            pltpu.SemaphoreType.DMA,
            pltpu.SemaphoreType.DMA,
            pltpu.SemaphoreType.DMA,
        ],
        compiler_params=pltpu.CompilerParams(
            use_tc_tiling_on_sc=False, needs_layout_passes=False
        ),
    )(idxt, table)
    return jnp.transpose(out, (2, 4, 0, 1, 3)).reshape(B, L, EMBED_DIM)


# speedup vs baseline: 1.3303x; 1.1477x over previous
import jax
import jax.numpy as jnp
from jax import lax
from jax.experimental import pallas as pl
from jax.experimental.pallas import tpu as pltpu
from jax.experimental.pallas import tpu_sc as plsc

B = 4096
L = 200
VOCAB = 1000000
EMBED_DIM = 64
PAD_DIM = 128

_info = plsc.get_sparse_core_info()
_NC = _info.num_cores
_NS = _info.num_subcores
_NW = _NC * _NS

_TC = 256
_NFULL = VOCAB // _TC
_KPW = _NFULL // _NW
_NEXTRA = _NFULL - _KPW * _NW
_TAIL = VOCAB - _NFULL * _TC


def _transpose_kernel(wt_hbm, out_hbm, in_v0, in_v1, out_v0, out_v1, in_t, out_t,
                      sem_i0, sem_i1, sem_o0, sem_o1, sem_t):
    in_v = (in_v0, in_v1)
    out_v = (out_v0, out_v1)
    sem_i = (sem_i0, sem_i1)
    sem_o = (sem_o0, sem_o1)
    wid = lax.axis_index("s") * _NC + lax.axis_index("c")

    iota16 = lax.iota(jnp.int32, 16)
    iota64 = iota16 * EMBED_DIM

    def in_start(start, b):
        pltpu.async_copy(wt_hbm.at[:, pl.ds(start, _TC)], in_v[b], sem_i[b])

    def in_wait(start, b):
        pltpu.make_async_copy(
            wt_hbm.at[:, pl.ds(start, _TC)], in_v[b], sem_i[b]
        ).wait()

    def out_start(start, b):
        pltpu.async_copy(
            out_v[b], out_hbm.at[pl.ds(start * EMBED_DIM, _TC * EMBED_DIM)],
            sem_o[b],
        )

    def out_wait(start, b):
        pltpu.make_async_copy(
            out_v[b], out_hbm.at[pl.ds(start * EMBED_DIM, _TC * EMBED_DIM)],
            sem_o[b],
        ).wait()

    def transpose_body(b):
        @pl.loop(0, _TC // 16)
        def _(tb):
            t0 = tb * 16
            col_idx = iota16 + t0
            for j2 in range(0, 16, 2):
                batch = []
                for j in (j2, j2 + 1):
                    rot_j = (iota16 + j) & 15
                    out_j = iota64 + rot_j
                    for d0 in range(0, EMBED_DIM, 16):
                        v = plsc.load_gather(
                            in_v[b].at[pl.ds(d0, 16)], [rot_j, col_idx]
                        )
                        batch.append((out_j + (t0 * EMBED_DIM + d0), v))
                for oidx, v in batch:
                    plsc.store_scatter(out_v[b], [oidx], v)

    def chunk_start(c):
        return c * _TC

    in_start(chunk_start(wid), 0)
    in_start(chunk_start(_NW + wid), 1)

    @pl.loop(0, _KPW - 2, step=2)
    def _(k):
        for b in range(2):
            c = (k + b) * _NW + wid
            start = chunk_start(c)
            in_wait(start, b)

            @pl.when(k + b >= 2)
            def _():
                out_wait(chunk_start((k + b - 2) * _NW + wid), b)

            transpose_body(b)
            out_start(start, b)
            in_start(chunk_start((k + b + 2) * _NW + wid), b)

    for b in range(2):
        c = (_KPW - 2 + b) * _NW + wid
        start = chunk_start(c)
        in_wait(start, b)
        out_wait(chunk_start((_KPW - 4 + b) * _NW + wid), b)
        transpose_body(b)
        out_start(start, b)

    @pl.when(wid < _NEXTRA)
    def _():
        start = chunk_start(_KPW * _NW + wid)
        in_start(start, 0)
        in_wait(start, 0)
        out_wait(chunk_start((_KPW - 2) * _NW + wid), 0)
        transpose_body(0)
        out_start(start, 0)
        out_wait(start, 0)

    @pl.when(wid >= _NEXTRA)
    def _():
        out_wait(chunk_start((_KPW - 2) * _NW + wid), 0)

    out_wait(chunk_start((_KPW - 1) * _NW + wid), 1)

    @pl.when(wid == _NEXTRA)
    def _():
        start = _NFULL * _TC
        pltpu.async_copy(wt_hbm.at[:, pl.ds(start, _TAIL)], in_t, sem_t)
        pltpu.make_async_copy(
            wt_hbm.at[:, pl.ds(start, _TAIL)], in_t, sem_t
        ).wait()

        @pl.loop(0, _TAIL // 16)
        def _(tb):
            t0 = tb * 16
            col_idx = iota16 + t0
            for j2 in range(0, 16, 2):
                batch = []
                for j in (j2, j2 + 1):
                    rot_j = (iota16 + j) & 15
                    out_j = iota64 + rot_j
                    for d0 in range(0, EMBED_DIM, 16):
                        v = plsc.load_gather(
                            in_t.at[pl.ds(d0, 16)], [rot_j, col_idx]
                        )
                        batch.append((out_j + (t0 * EMBED_DIM + d0), v))
                for oidx, v in batch:
                    plsc.store_scatter(out_t, [oidx], v)

        pltpu.async_copy(
            out_t, out_hbm.at[pl.ds(start * EMBED_DIM, _TAIL * EMBED_DIM)], sem_t
        )
        pltpu.make_async_copy(
            out_t, out_hbm.at[pl.ds(start * EMBED_DIM, _TAIL * EMBED_DIM)], sem_t
        ).wait()


_N = B * L
_TB = 128
_NBB = B // _TB
_NTASK = L * _NBB
_TPW = _NTASK // _NW


def _gather_t_kernel(idxt_hbm, table_hbm, out_hbm,
                     idx_all, rows_v0, rows_v1, rows_v2, rows_v3,
                     slab_v0, slab_v1, slab_v2, slab_v3,
                     sem_g0, sem_g1, sem_g2, sem_g3,
                     sem_o0, sem_o1, sem_o2, sem_o3):
    rows_v = (rows_v0, rows_v1, rows_v2, rows_v3)
    slab_v = (slab_v0, slab_v1, slab_v2, slab_v3)
    sem_g = (sem_g0, sem_g1, sem_g2, sem_g3)
    sem_o = (sem_o0, sem_o1, sem_o2, sem_o3)
    wid = lax.axis_index("s") * _NC + lax.axis_index("c")
    iota16 = lax.iota(jnp.int32, 16)

    pltpu.sync_copy(
        idxt_hbm.at[pl.ds(wid * _TPW * _TB, _TPW * _TB)], idx_all
    )

    def gather_start(k, b):
        pltpu.async_copy(
            table_hbm.at[idx_all.at[pl.ds(k * _TB, _TB)]], rows_v[b], sem_g[b]
        )

    def gather_wait(k, b):
        pltpu.make_async_copy(
            table_hbm.at[idx_all.at[pl.ds(k * _TB, _TB)]], rows_v[b], sem_g[b]
        ).wait()

    def out_ref(t):
        return out_hbm.at[t // _NBB, :, t % _NBB]

    def out_start(t, b):
        pltpu.async_copy(slab_v[b], out_ref(t), sem_o[b])

    def out_wait(t, b):
        pltpu.make_async_copy(slab_v[b], out_ref(t), sem_o[b]).wait()

    def transpose_body(b):
        @pl.loop(0, _TB // 16)
        def _(tb):
            t_idx = iota16 + tb * 16
            for j2 in range(0, 16, 2):
                batch = []
                for j in (j2, j2 + 1):
                    rot_j = (iota16 + j) & 15
                    db_j = rot_j >> 3
                    d8_j = rot_j & 7
                    for d0 in range(0, EMBED_DIM, 16):
                        v = plsc.load_gather(rows_v[b], [t_idx, rot_j + d0])
                        batch.append((db_j + (d0 >> 3), d8_j, v))
                for idx_db, idx_d8, v in batch:
                    plsc.store_scatter(slab_v[b], [idx_db, idx_d8, t_idx], v)

    base = wid * _TPW
    for b in range(4):
        gather_start(b, b)

    @pl.loop(0, _TPW - 4, step=4)
    def _(k):
        for b in range(4):
            t = base + k + b
            gather_wait(k + b, b)

            @pl.when(k + b >= 4)
            def _():
                out_wait(t - 4, b)

            transpose_body(b)
            out_start(t, b)
            gather_start(k + b + 4, b)

    for b in range(4):
        t = base + _TPW - 4 + b
        gather_wait(_TPW - 4 + b, b)
        out_wait(t - 4, b)
        transpose_body(b)
        out_start(t, b)
    for b in range(4):
        out_wait(base + _TPW - 4 + b, b)


@jax.jit
def kernel(tokens, word_embed_weight):
    mesh = plsc.VectorSubcoreMesh(core_axis_name="c", subcore_axis_name="s")

    wt = word_embed_weight.T
    tlin = pl.kernel(
        _transpose_kernel,
        out_type=jax.ShapeDtypeStruct((VOCAB * EMBED_DIM,), jnp.float32),
        mesh=mesh,
        scratch_types=[
            pltpu.VMEM((EMBED_DIM, _TC), jnp.float32),
            pltpu.VMEM((EMBED_DIM, _TC), jnp.float32),
            pltpu.VMEM((_TC * EMBED_DIM,), jnp.float32),
            pltpu.VMEM((_TC * EMBED_DIM,), jnp.float32),
            pltpu.VMEM((EMBED_DIM, _TAIL), jnp.float32),
            pltpu.VMEM((_TAIL * EMBED_DIM,), jnp.float32),
            pltpu.SemaphoreType.DMA,
            pltpu.SemaphoreType.DMA,
            pltpu.SemaphoreType.DMA,
            pltpu.SemaphoreType.DMA,
            pltpu.SemaphoreType.DMA,
        ],
        compiler_params=pltpu.CompilerParams(
            use_tc_tiling_on_sc=True, needs_layout_passes=False
        ),
    )(wt)
    table = tlin.reshape(VOCAB, EMBED_DIM)

    idxt = tokens.T.reshape(_N).astype(jnp.int32)
    out = pl.kernel(
        _gather_t_kernel,
        out_type=jax.ShapeDtypeStruct(
            (L, EMBED_DIM // 8, B // _TB, 8, _TB), jnp.float32
        ),
        mesh=mesh,
        scratch_types=[
            pltpu.VMEM((_TPW * _TB,), jnp.int32),
            pltpu.VMEM((_TB, EMBED_DIM), jnp.float32),
            pltpu.VMEM((_TB, EMBED_DIM), jnp.float32),
            pltpu.VMEM((_TB, EMBED_DIM), jnp.float32),
            pltpu.VMEM((_TB, EMBED_DIM), jnp.float32),
            pltpu.VMEM((EMBED_DIM // 8, 8, _TB), jnp.float32),
            pltpu.VMEM((EMBED_DIM // 8, 8, _TB), jnp.float32),
            pltpu.VMEM((EMBED_DIM // 8, 8, _TB), jnp.float32),
            pltpu.VMEM((EMBED_DIM // 8, 8, _TB), jnp.float32),
            pltpu.SemaphoreType.DMA,
            pltpu.SemaphoreType.DMA,
            pltpu.SemaphoreType.DMA,
            pltpu.SemaphoreType.DMA,
            pltpu.SemaphoreType.DMA,
            pltpu.SemaphoreType.DMA,
            pltpu.SemaphoreType.DMA,
            pltpu.SemaphoreType.DMA,
        ],
        compiler_params=pltpu.CompilerParams(
            use_tc_tiling_on_sc=False, needs_layout_passes=False
        ),
    )(idxt, table)
    return jnp.transpose(out, (2, 4, 0, 1, 3)).reshape(B, L, EMBED_DIM)


# speedup vs baseline: 1.3368x; 1.0049x over previous
import jax
import jax.numpy as jnp
from jax import lax
from jax.experimental import pallas as pl
from jax.experimental.pallas import tpu as pltpu
from jax.experimental.pallas import tpu_sc as plsc

B = 4096
L = 200
VOCAB = 1000000
EMBED_DIM = 64
PAD_DIM = 128

_info = plsc.get_sparse_core_info()
_NC = _info.num_cores
_NS = _info.num_subcores
_NW = _NC * _NS

_TC = 256
_NFULL = VOCAB // _TC
_KPW = _NFULL // _NW
_NEXTRA = _NFULL - _KPW * _NW
_TAIL = VOCAB - _NFULL * _TC


def _transpose_kernel(wt_hbm, out_hbm, in_v0, in_v1, out_v0, out_v1, in_t, out_t,
                      sem_i0, sem_i1, sem_o0, sem_o1, sem_t):
    in_v = (in_v0, in_v1)
    out_v = (out_v0, out_v1)
    sem_i = (sem_i0, sem_i1)
    sem_o = (sem_o0, sem_o1)
    wid = lax.axis_index("s") * _NC + lax.axis_index("c")

    iota16 = lax.iota(jnp.int32, 16)
    iota64 = iota16 * EMBED_DIM

    def in_start(start, b):
        pltpu.async_copy(wt_hbm.at[:, pl.ds(start, _TC)], in_v[b], sem_i[b])

    def in_wait(start, b):
        pltpu.make_async_copy(
            wt_hbm.at[:, pl.ds(start, _TC)], in_v[b], sem_i[b]
        ).wait()

    def out_start(start, b):
        pltpu.async_copy(
            out_v[b], out_hbm.at[pl.ds(start * EMBED_DIM, _TC * EMBED_DIM)],
            sem_o[b],
        )

    def out_wait(start, b):
        pltpu.make_async_copy(
            out_v[b], out_hbm.at[pl.ds(start * EMBED_DIM, _TC * EMBED_DIM)],
            sem_o[b],
        ).wait()

    def transpose_body(b):
        @pl.loop(0, _TC // 16)
        def _(tb):
            t0 = tb * 16
            col_idx = iota16 + t0
            for j2 in range(0, 16, 2):
                batch = []
                for j in (j2, j2 + 1):
                    rot_j = (iota16 + j) & 15
                    out_j = iota64 + rot_j
                    for d0 in range(0, EMBED_DIM, 16):
                        v = plsc.load_gather(
                            in_v[b].at[pl.ds(d0, 16)], [rot_j, col_idx]
                        )
                        batch.append((out_j + (t0 * EMBED_DIM + d0), v))
                for oidx, v in batch:
                    plsc.store_scatter(out_v[b], [oidx], v)

    def chunk_start(c):
        return c * _TC

    in_start(chunk_start(wid), 0)
    in_start(chunk_start(_NW + wid), 1)

    @pl.loop(0, _KPW - 2, step=2)
    def _(k):
        for b in range(2):
            c = (k + b) * _NW + wid
            start = chunk_start(c)
            in_wait(start, b)

            @pl.when(k + b >= 2)
            def _():
                out_wait(chunk_start((k + b - 2) * _NW + wid), b)

            transpose_body(b)
            out_start(start, b)
            in_start(chunk_start((k + b + 2) * _NW + wid), b)

    for b in range(2):
        c = (_KPW - 2 + b) * _NW + wid
        start = chunk_start(c)
        in_wait(start, b)
        out_wait(chunk_start((_KPW - 4 + b) * _NW + wid), b)
        transpose_body(b)
        out_start(start, b)

    @pl.when(wid < _NEXTRA)
    def _():
        start = chunk_start(_KPW * _NW + wid)
        in_start(start, 0)
        in_wait(start, 0)
        out_wait(chunk_start((_KPW - 2) * _NW + wid), 0)
        transpose_body(0)
        out_start(start, 0)
        out_wait(start, 0)

    @pl.when(wid >= _NEXTRA)
    def _():
        out_wait(chunk_start((_KPW - 2) * _NW + wid), 0)

    out_wait(chunk_start((_KPW - 1) * _NW + wid), 1)

    @pl.when(wid == _NEXTRA)
    def _():
        start = _NFULL * _TC
        pltpu.async_copy(wt_hbm.at[:, pl.ds(start, _TAIL)], in_t, sem_t)
        pltpu.make_async_copy(
            wt_hbm.at[:, pl.ds(start, _TAIL)], in_t, sem_t
        ).wait()

        @pl.loop(0, _TAIL // 16)
        def _(tb):
            t0 = tb * 16
            col_idx = iota16 + t0
            for j2 in range(0, 16, 2):
                batch = []
                for j in (j2, j2 + 1):
                    rot_j = (iota16 + j) & 15
                    out_j = iota64 + rot_j
                    for d0 in range(0, EMBED_DIM, 16):
                        v = plsc.load_gather(
                            in_t.at[pl.ds(d0, 16)], [rot_j, col_idx]
                        )
                        batch.append((out_j + (t0 * EMBED_DIM + d0), v))
                for oidx, v in batch:
                    plsc.store_scatter(out_t, [oidx], v)

        pltpu.async_copy(
            out_t, out_hbm.at[pl.ds(start * EMBED_DIM, _TAIL * EMBED_DIM)], sem_t
        )
        pltpu.make_async_copy(
            out_t, out_hbm.at[pl.ds(start * EMBED_DIM, _TAIL * EMBED_DIM)], sem_t
        ).wait()


_N = B * L
_TB = 128
_NBB = B // _TB
_NTASK = L * _NBB
_TPW = _NTASK // _NW


def _gather_t_kernel(idxt_hbm, table_hbm, out_hbm,
                     idx_all, rows_v0, rows_v1, rows_v2, rows_v3, rows_v4,
                     slab_v0, slab_v1, slab_v2, slab_v3, slab_v4,
                     sem_g0, sem_g1, sem_g2, sem_g3, sem_g4,
                     sem_o0, sem_o1, sem_o2, sem_o3, sem_o4):
    rows_v = (rows_v0, rows_v1, rows_v2, rows_v3, rows_v4)
    slab_v = (slab_v0, slab_v1, slab_v2, slab_v3, slab_v4)
    sem_g = (sem_g0, sem_g1, sem_g2, sem_g3, sem_g4)
    sem_o = (sem_o0, sem_o1, sem_o2, sem_o3, sem_o4)
    wid = lax.axis_index("s") * _NC + lax.axis_index("c")
    iota16 = lax.iota(jnp.int32, 16)

    pltpu.sync_copy(
        idxt_hbm.at[pl.ds(wid * _TPW * _TB, _TPW * _TB)], idx_all
    )

    def gather_start(k, b):
        pltpu.async_copy(
            table_hbm.at[idx_all.at[pl.ds(k * _TB, _TB)]], rows_v[b], sem_g[b]
        )

    def gather_wait(k, b):
        pltpu.make_async_copy(
            table_hbm.at[idx_all.at[pl.ds(k * _TB, _TB)]], rows_v[b], sem_g[b]
        ).wait()

    def out_ref(t):
        return out_hbm.at[t // _NBB, :, t % _NBB]

    def out_start(t, b):
        pltpu.async_copy(slab_v[b], out_ref(t), sem_o[b])

    def out_wait(t, b):
        pltpu.make_async_copy(slab_v[b], out_ref(t), sem_o[b]).wait()

    def transpose_body(b):
        @pl.loop(0, _TB // 16)
        def _(tb):
            t_idx = iota16 + tb * 16
            for j2 in range(0, 16, 2):
                batch = []
                for j in (j2, j2 + 1):
                    rot_j = (iota16 + j) & 15
                    db_j = rot_j >> 3
                    d8_j = rot_j & 7
                    for d0 in range(0, EMBED_DIM, 16):
                        v = plsc.load_gather(rows_v[b], [t_idx, rot_j + d0])
                        batch.append((db_j + (d0 >> 3), d8_j, v))
                for idx_db, idx_d8, v in batch:
                    plsc.store_scatter(slab_v[b], [idx_db, idx_d8, t_idx], v)

    base = wid * _TPW
    for b in range(5):
        gather_start(b, b)

    @pl.loop(0, _TPW - 5, step=5)
    def _(k):
        for b in range(5):
            t = base + k + b
            gather_wait(k + b, b)

            @pl.when(k + b >= 5)
            def _():
                out_wait(t - 5, b)

            transpose_body(b)
            out_start(t, b)
            gather_start(k + b + 5, b)

    for b in range(5):
        t = base + _TPW - 5 + b
        gather_wait(_TPW - 5 + b, b)
        out_wait(t - 5, b)
        transpose_body(b)
        out_start(t, b)
    for b in range(5):
        out_wait(base + _TPW - 5 + b, b)


@jax.jit
def kernel(tokens, word_embed_weight):
    mesh = plsc.VectorSubcoreMesh(core_axis_name="c", subcore_axis_name="s")

    wt = word_embed_weight.T
    tlin = pl.kernel(
        _transpose_kernel,
        out_type=jax.ShapeDtypeStruct((VOCAB * EMBED_DIM,), jnp.float32),
        mesh=mesh,
        scratch_types=[
            pltpu.VMEM((EMBED_DIM, _TC), jnp.float32),
            pltpu.VMEM((EMBED_DIM, _TC), jnp.float32),
            pltpu.VMEM((_TC * EMBED_DIM,), jnp.float32),
            pltpu.VMEM((_TC * EMBED_DIM,), jnp.float32),
            pltpu.VMEM((EMBED_DIM, _TAIL), jnp.float32),
            pltpu.VMEM((_TAIL * EMBED_DIM,), jnp.float32),
            pltpu.SemaphoreType.DMA,
            pltpu.SemaphoreType.DMA,
            pltpu.SemaphoreType.DMA,
            pltpu.SemaphoreType.DMA,
            pltpu.SemaphoreType.DMA,
        ],
        compiler_params=pltpu.CompilerParams(
            use_tc_tiling_on_sc=True, needs_layout_passes=False
        ),
    )(wt)
    table = tlin.reshape(VOCAB, EMBED_DIM)

    idxt = tokens.T.reshape(_N).astype(jnp.int32)
    out = pl.kernel(
        _gather_t_kernel,
        out_type=jax.ShapeDtypeStruct(
            (L, EMBED_DIM // 8, B // _TB, 8, _TB), jnp.float32
        ),
        mesh=mesh,
        scratch_types=[
            pltpu.VMEM((_TPW * _TB,), jnp.int32),
            pltpu.VMEM((_TB, EMBED_DIM), jnp.float32),
            pltpu.VMEM((_TB, EMBED_DIM), jnp.float32),
            pltpu.VMEM((_TB, EMBED_DIM), jnp.float32),
            pltpu.VMEM((_TB, EMBED_DIM), jnp.float32),
            pltpu.VMEM((_TB, EMBED_DIM), jnp.float32),
            pltpu.VMEM((EMBED_DIM // 8, 8, _TB), jnp.float32),
            pltpu.VMEM((EMBED_DIM // 8, 8, _TB), jnp.float32),
            pltpu.VMEM((EMBED_DIM // 8, 8, _TB), jnp.float32),
            pltpu.VMEM((EMBED_DIM // 8, 8, _TB), jnp.float32),
            pltpu.VMEM((EMBED_DIM // 8, 8, _TB), jnp.float32),
            pltpu.SemaphoreType.DMA,
            pltpu.SemaphoreType.DMA,
            pltpu.SemaphoreType.DMA,
            pltpu.SemaphoreType.DMA,
            pltpu.SemaphoreType.DMA,
            pltpu.SemaphoreType.DMA,
            pltpu.SemaphoreType.DMA,
            pltpu.SemaphoreType.DMA,
            pltpu.SemaphoreType.DMA,
            pltpu.SemaphoreType.DMA,
        ],
        compiler_params=pltpu.CompilerParams(
            use_tc_tiling_on_sc=False, needs_layout_passes=False
        ),
    )(idxt, table)
    return jnp.transpose(out, (2, 4, 0, 1, 3)).reshape(B, L, EMBED_DIM)
